# baseline (speedup 1.0000x reference)
import numpy as np
import jax
import jax.numpy as jnp
from jax.experimental import pallas as pl
from jax.experimental.pallas import tpu as pltpu

_H, _W = 128, 128
_N = _H * _W
_F = 128
_B = 16
_N_PATCHES = 8
_SEED = 42
_MIN_PATCH = (16, 16)
_MAX_PATCH = (32, 32)


def _static_patch_coords():
    rng = np.random.default_rng(_SEED)
    coords = []
    for _ in range(_N_PATCHES):
        upper_bound = [s - p for s, p in zip((_H, _W), _MAX_PATCH)]
        lower = np.array([rng.integers(0, i) for i in upper_bound])
        ps = np.array([rng.integers(m, M) for m, M in zip(_MIN_PATCH, _MAX_PATCH)])
        upper = lower + ps
        coords.append((int(lower[0]), int(lower[1]), int(upper[0]), int(upper[1])))
    return coords


_COORDS = _static_patch_coords()

_MASK_NP = np.zeros((_H, _W, 1), dtype=np.float32)
for _r0, _c0, _r1, _c1 in _COORDS:
    _MASK_NP[_r0:_r1, _c0:_c1, 0] = 1.0


def _body(x_ref, mv_ref, pos_ref, m_ref, o_ref, *patch_refs):
    x = x_ref[0]
    repl = pos_ref[...] + mv_ref[0, 0][None, None, :]
    o_ref[0] = jnp.where(m_ref[...] > 0.0, repl, x)

    for i, (r0, c0, r1, c1) in enumerate(_COORDS):
        patch_refs[i][0] = x[r0:r1, c0:c1, :]


@jax.jit
def kernel(X, mask_vector, positional_embedding):
    X4 = X.reshape(_B, _H, _W, _F)
    mv = mask_vector.reshape(1, 1, _F)
    pos3 = positional_embedding.reshape(_H, _W, _F)
    mask = jnp.asarray(_MASK_NP)

    out_shapes = [jax.ShapeDtypeStruct((_B, _H, _W, _F), jnp.float32)]
    out_specs = [pl.BlockSpec((1, _H, _W, _F), lambda b: (b, 0, 0, 0))]
    for (r0, c0, r1, c1) in _COORDS:
        ph, pw = r1 - r0, c1 - c0
        out_shapes.append(jax.ShapeDtypeStruct((_B, ph, pw, _F), jnp.float32))
        out_specs.append(
            pl.BlockSpec((1, ph, pw, _F), lambda b: (b, 0, 0, 0))
        )

    outs = pl.pallas_call(
        _body,
        grid=(_B,),
        in_specs=[
            pl.BlockSpec((1, _H, _W, _F), lambda b: (b, 0, 0, 0)),
            pl.BlockSpec(memory_space=pltpu.MemorySpace.VMEM),
            pl.BlockSpec(memory_space=pltpu.MemorySpace.VMEM),
            pl.BlockSpec(memory_space=pltpu.MemorySpace.VMEM),
        ],
        out_specs=out_specs,
        out_shape=out_shapes,
    )(X4, mv, pos3, mask)

    Xm = outs[0].reshape(_B, _N, _F)
    patches = tuple(
        p.reshape(_B, p.shape[1] * p.shape[2], _F) for p in outs[1:]
    )
    return (Xm,) + patches
